# baseline (speedup 1.0000x reference)
import functools

import jax
import jax.numpy as jnp
from jax import lax
from jax.experimental import pallas as pl
from jax.experimental.pallas import tpu as pltpu
from jax.experimental.pallas import tpu_sc as plsc

_NUM_ENT = 100000
_NUM_REL = 1000
_HIDDEN = 32
_BATCH = 1024
_BN = 2048


def _sc_gather_all(e3, r3, emapT, wT, relT):
    info = plsc.get_sparse_core_info()
    nc, ns, lanes = info.num_cores, info.num_subcores, info.num_lanes
    nw = nc * ns
    bpw = _BATCH // nw
    nbuf = 8
    mesh = plsc.VectorSubcoreMesh(core_axis_name="c", subcore_axis_name="s")

    @functools.partial(
        pl.kernel,
        mesh=mesh,
        out_type=jax.ShapeDtypeStruct((nw, bpw, 3 * _HIDDEN + 1), jnp.float32),
        scratch_types=[
            pltpu.VMEM((1, bpw), jnp.int32),
            pltpu.VMEM((1, bpw), jnp.int32),
            [pltpu.VMEM((2, 128), jnp.int32) for _ in range(nbuf)],
            [pltpu.VMEM((_HIDDEN, 128), jnp.float32) for _ in range(nbuf)],
            pltpu.VMEM((_HIDDEN, _NUM_REL), jnp.float32),
            pltpu.VMEM((bpw, 3 * _HIDDEN + 1), jnp.float32),
            pltpu.SemaphoreType.DMA,
            pltpu.SemaphoreType.DMA,
            pltpu.SemaphoreType.DMA,
        ],
        compiler_params=pltpu.CompilerParams(
            use_tc_tiling_on_sc=True, needs_layout_passes=False),
    )
    def gather_k(e_hbm, r_hbm, map_hbm, wt_hbm, rel_hbm, mt_out,
                 ev, rv, mbufs, wbufs, rtab, slab_v, msem, wsem, rsem):
        wid = lax.axis_index("s") * nc + lax.axis_index("c")
        iota = lax.iota(jnp.int32, lanes)
        ones16 = jnp.full((lanes,), 1.0, jnp.float32)

        rel_cp = pltpu.async_copy(rel_hbm, rtab, rsem)
        pltpu.sync_copy(e_hbm.at[wid], ev)
        pltpu.sync_copy(r_hbm.at[wid], rv)
        e_vecs = [ev[0, pl.ds(k * lanes, lanes)]
                  for k in range(bpw // lanes)]
        r_vecs = [rv[0, pl.ds(k * lanes, lanes)]
                  for k in range(bpw // lanes)]

        rows01 = iota % 2
        mcopies = [None] * bpw
        mlanes = [None] * bpw
        wids = [None] * (2 * bpw)

        def m_issue(i):
            e = e_vecs[i // lanes][i % lanes]
            et = pl.multiple_of((e // 128) * 128, 128)
            mlanes[i] = e - et
            mcopies[i] = pltpu.async_copy(
                map_hbm.at[:, pl.ds(et, 128)], mbufs[i % nbuf], msem)

        def m_extract(i):
            mcopies[i].wait()
            vals = plsc.load_gather(
                mbufs[i % nbuf],
                [rows01, jnp.full((lanes,), 0, jnp.int32) + mlanes[i]])
            wids[i] = vals[0]
            wids[bpw + i] = vals[1]

        ncols = 2 * bpw
        wcopies = [None] * ncols
        wlanes = [None] * ncols

        def w_issue(k, j):
            jt = pl.multiple_of((j // 128) * 128, 128)
            wlanes[k] = j - jt
            wcopies[k] = pltpu.async_copy(
                wt_hbm.at[:, pl.ds(jt, 128)], wbufs[k % nbuf], wsem)

        def w_extract(k):
            wcopies[k].wait()
            row0 = 0 if k % 2 == 0 else _HIDDEN
            col = jnp.full((lanes,), k // 2, jnp.int32)
            lane_idx = jnp.full((lanes,), 0, jnp.int32) + wlanes[k]
            for h in range(_HIDDEN // lanes):
                vals = plsc.load_gather(
                    wbufs[k % nbuf], [iota + h * lanes, lane_idx])
                plsc.store_scatter(
                    slab_v, [col, iota + (row0 + h * lanes)], vals)

        for i in range(min(nbuf, bpw)):
            m_issue(i)
        for i in range(bpw):
            m_extract(i)
            if i + nbuf < bpw:
                m_issue(i + nbuf)
            for k in (2 * i, 2 * i + 1):
                if k >= nbuf:
                    w_extract(k - nbuf)
                w_issue(k, wids[i] if k % 2 == 0 else wids[bpw + i])
        for k in range(ncols - nbuf, ncols):
            w_extract(k)

        rel_cp.wait()
        for i in range(bpw):
            rid = r_vecs[i // lanes][i % lanes]
            col = jnp.full((lanes,), i, jnp.int32)
            lane_idx = jnp.full((lanes,), 0, jnp.int32) + rid
            for h in range(_HIDDEN // lanes):
                vals = plsc.load_gather(rtab, [iota + h * lanes, lane_idx])
                plsc.store_scatter(
                    slab_v, [col, iota + (2 * _HIDDEN + h * lanes)], vals)

        ones_col = jnp.full((lanes,), 3 * _HIDDEN, jnp.int32)
        for k in range(bpw // lanes):
            plsc.store_scatter(slab_v, [iota + k * lanes, ones_col], ones16)

        pltpu.sync_copy(slab_v, mt_out.at[wid])

    return gather_k(e3, r3, emapT, wT, relT)


def _mm_body(wt_ref, b_ref, mt_ref, out_ref):
    lhs = jnp.concatenate([wt_ref[...], b_ref[...]], axis=0)
    out_ref[...] = lax.dot_general(
        lhs, mt_ref[...], (((0,), (0,)), ((), ())),
        preferred_element_type=jnp.float32)


def _tc_project(WT, b2d, mt_aug):
    grid = (pl.cdiv(_NUM_ENT, _BN),)
    outT = pl.pallas_call(
        _mm_body,
        grid=grid,
        in_specs=[
            pl.BlockSpec((3 * _HIDDEN, _BN), lambda i: (0, i)),
            pl.BlockSpec((1, _BN), lambda i: (0, i)),
            pl.BlockSpec((3 * _HIDDEN + 1, _BATCH), lambda i: (0, 0)),
        ],
        out_specs=pl.BlockSpec((_BN, _BATCH), lambda i: (i, 0)),
        out_shape=jax.ShapeDtypeStruct((_NUM_ENT, _BATCH), jnp.float32),
        compiler_params=pltpu.CompilerParams(
            dimension_semantics=("arbitrary",)),
    )(WT, b2d, mt_aug)
    return outT.T


def kernel(batch_data, ent_word_map, words_embd, rel_embed, W, b):
    nw = 32
    bpw = _BATCH // nw
    e3 = batch_data[:, 0].reshape(nw, 1, bpw)
    r3 = batch_data[:, 1].reshape(nw, 1, bpw)
    m3 = _sc_gather_all(e3, r3, ent_word_map.T, words_embd.T, rel_embed.T)
    mt_aug = m3.reshape(_BATCH, 3 * _HIDDEN + 1).T
    return _tc_project(W.T, b.reshape(1, -1), mt_aug)


# speedup vs baseline: 1.0058x; 1.0058x over previous
import functools

import jax
import jax.numpy as jnp
from jax import lax
from jax.experimental import pallas as pl
from jax.experimental.pallas import tpu as pltpu
from jax.experimental.pallas import tpu_sc as plsc

_NUM_ENT = 100000
_NUM_REL = 1000
_HIDDEN = 32
_BATCH = 1024
_BN = 4096


def _sc_gather_all(e3, r3, emapT, wT, relT):
    info = plsc.get_sparse_core_info()
    nc, ns, lanes = info.num_cores, info.num_subcores, info.num_lanes
    nw = nc * ns
    bpw = _BATCH // nw
    nbuf = 16
    mesh = plsc.VectorSubcoreMesh(core_axis_name="c", subcore_axis_name="s")

    @functools.partial(
        pl.kernel,
        mesh=mesh,
        out_type=jax.ShapeDtypeStruct((nw, bpw, 3 * _HIDDEN + 1), jnp.float32),
        scratch_types=[
            pltpu.VMEM((1, bpw), jnp.int32),
            pltpu.VMEM((1, bpw), jnp.int32),
            [pltpu.VMEM((2, 128), jnp.int32) for _ in range(nbuf)],
            [pltpu.VMEM((_HIDDEN, 128), jnp.float32) for _ in range(nbuf)],
            pltpu.VMEM((_HIDDEN, _NUM_REL), jnp.float32),
            pltpu.VMEM((bpw, 3 * _HIDDEN + 1), jnp.float32),
            pltpu.SemaphoreType.DMA,
            pltpu.SemaphoreType.DMA,
            pltpu.SemaphoreType.DMA,
        ],
        compiler_params=pltpu.CompilerParams(
            use_tc_tiling_on_sc=True, needs_layout_passes=False),
    )
    def gather_k(e_hbm, r_hbm, map_hbm, wt_hbm, rel_hbm, mt_out,
                 ev, rv, mbufs, wbufs, rtab, slab_v, msem, wsem, rsem):
        wid = lax.axis_index("s") * nc + lax.axis_index("c")
        iota = lax.iota(jnp.int32, lanes)
        ones16 = jnp.full((lanes,), 1.0, jnp.float32)

        rel_cp = pltpu.async_copy(rel_hbm, rtab, rsem)
        pltpu.sync_copy(e_hbm.at[wid], ev)
        pltpu.sync_copy(r_hbm.at[wid], rv)
        e_vecs = [ev[0, pl.ds(k * lanes, lanes)]
                  for k in range(bpw // lanes)]
        r_vecs = [rv[0, pl.ds(k * lanes, lanes)]
                  for k in range(bpw // lanes)]

        rows01 = iota % 2
        mcopies = [None] * bpw
        mlanes = [None] * bpw
        wids = [None] * (2 * bpw)

        def m_issue(i):
            e = e_vecs[i // lanes][i % lanes]
            et = pl.multiple_of((e // 128) * 128, 128)
            mlanes[i] = e - et
            mcopies[i] = pltpu.async_copy(
                map_hbm.at[:, pl.ds(et, 128)], mbufs[i % nbuf], msem)

        def m_extract(i):
            mcopies[i].wait()
            vals = plsc.load_gather(
                mbufs[i % nbuf],
                [rows01, jnp.full((lanes,), 0, jnp.int32) + mlanes[i]])
            wids[i] = vals[0]
            wids[bpw + i] = vals[1]

        ncols = 2 * bpw
        wcopies = [None] * ncols
        wlanes = [None] * ncols

        def w_issue(k, j):
            jt = pl.multiple_of((j // 128) * 128, 128)
            wlanes[k] = j - jt
            wcopies[k] = pltpu.async_copy(
                wt_hbm.at[:, pl.ds(jt, 128)], wbufs[k % nbuf], wsem)

        def w_extract(k):
            wcopies[k].wait()
            row0 = 0 if k % 2 == 0 else _HIDDEN
            col = jnp.full((lanes,), k // 2, jnp.int32)
            lane_idx = jnp.full((lanes,), 0, jnp.int32) + wlanes[k]
            for h in range(_HIDDEN // lanes):
                vals = plsc.load_gather(
                    wbufs[k % nbuf], [iota + h * lanes, lane_idx])
                plsc.store_scatter(
                    slab_v, [col, iota + (row0 + h * lanes)], vals)

        for i in range(min(nbuf, bpw)):
            m_issue(i)
        for i in range(bpw):
            m_extract(i)
            if i + nbuf < bpw:
                m_issue(i + nbuf)
            for k in (2 * i, 2 * i + 1):
                if k >= nbuf:
                    w_extract(k - nbuf)
                w_issue(k, wids[i] if k % 2 == 0 else wids[bpw + i])
        for k in range(ncols - nbuf, ncols):
            w_extract(k)

        rel_cp.wait()
        for i in range(bpw):
            rid = r_vecs[i // lanes][i % lanes]
            col = jnp.full((lanes,), i, jnp.int32)
            lane_idx = jnp.full((lanes,), 0, jnp.int32) + rid
            for h in range(_HIDDEN // lanes):
                vals = plsc.load_gather(rtab, [iota + h * lanes, lane_idx])
                plsc.store_scatter(
                    slab_v, [col, iota + (2 * _HIDDEN + h * lanes)], vals)

        ones_col = jnp.full((lanes,), 3 * _HIDDEN, jnp.int32)
        for k in range(bpw // lanes):
            plsc.store_scatter(slab_v, [iota + k * lanes, ones_col], ones16)

        pltpu.sync_copy(slab_v, mt_out.at[wid])

    return gather_k(e3, r3, emapT, wT, relT)


def _mm_body(wt_ref, b_ref, mt_ref, out_ref):
    lhs = jnp.concatenate([wt_ref[...], b_ref[...]], axis=0)
    out_ref[...] = lax.dot_general(
        lhs, mt_ref[...], (((0,), (0,)), ((), ())),
        preferred_element_type=jnp.float32)


def _tc_project(WT, b2d, mt_aug):
    grid = (pl.cdiv(_NUM_ENT, _BN),)
    outT = pl.pallas_call(
        _mm_body,
        grid=grid,
        in_specs=[
            pl.BlockSpec((3 * _HIDDEN, _BN), lambda i: (0, i)),
            pl.BlockSpec((1, _BN), lambda i: (0, i)),
            pl.BlockSpec((3 * _HIDDEN + 1, _BATCH), lambda i: (0, 0)),
        ],
        out_specs=pl.BlockSpec((_BN, _BATCH), lambda i: (i, 0)),
        out_shape=jax.ShapeDtypeStruct((_NUM_ENT, _BATCH), jnp.float32),
        compiler_params=pltpu.CompilerParams(
            dimension_semantics=("arbitrary",)),
    )(WT, b2d, mt_aug)
    return outT.T


def kernel(batch_data, ent_word_map, words_embd, rel_embed, W, b):
    nw = 32
    bpw = _BATCH // nw
    e3 = batch_data[:, 0].reshape(nw, 1, bpw)
    r3 = batch_data[:, 1].reshape(nw, 1, bpw)
    m3 = _sc_gather_all(e3, r3, ent_word_map.T, words_embd.T, rel_embed.T)
    mt_aug = m3.reshape(_BATCH, 3 * _HIDDEN + 1).T
    return _tc_project(W.T, b.reshape(1, -1), mt_aug)


# speedup vs baseline: 1.0133x; 1.0075x over previous
import functools

import jax
import jax.numpy as jnp
from jax import lax
from jax.experimental import pallas as pl
from jax.experimental.pallas import tpu as pltpu
from jax.experimental.pallas import tpu_sc as plsc

_NUM_ENT = 100000
_NUM_REL = 1000
_HIDDEN = 32
_BATCH = 1024
_BN = 4096


def _sc_gather_all(e3, r3, emapT, wT, relT):
    info = plsc.get_sparse_core_info()
    nc, ns, lanes = info.num_cores, info.num_subcores, info.num_lanes
    nw = nc * ns
    bpw = _BATCH // nw
    nbuf = 8
    mesh = plsc.VectorSubcoreMesh(core_axis_name="c", subcore_axis_name="s")

    @functools.partial(
        pl.kernel,
        mesh=mesh,
        out_type=jax.ShapeDtypeStruct((nw, bpw, 3 * _HIDDEN + 1), jnp.float32),
        scratch_types=[
            pltpu.VMEM((1, bpw), jnp.int32),
            pltpu.VMEM((1, bpw), jnp.int32),
            [pltpu.VMEM((2, 128), jnp.int32) for _ in range(nbuf)],
            [pltpu.VMEM((_HIDDEN, 128), jnp.float32) for _ in range(nbuf)],
            pltpu.VMEM((_HIDDEN, _NUM_REL), jnp.float32),
            pltpu.VMEM((bpw, 3 * _HIDDEN + 1), jnp.float32),
            pltpu.SemaphoreType.DMA,
            pltpu.SemaphoreType.DMA,
            pltpu.SemaphoreType.DMA,
        ],
        compiler_params=pltpu.CompilerParams(
            use_tc_tiling_on_sc=True, needs_layout_passes=False),
    )
    def gather_k(e_hbm, r_hbm, map_hbm, wt_hbm, rel_hbm, mt_out,
                 ev, rv, mbufs, wbufs, rtab, slab_v, msem, wsem, rsem):
        wid = lax.axis_index("s") * nc + lax.axis_index("c")
        iota = lax.iota(jnp.int32, lanes)
        ones16 = jnp.full((lanes,), 1.0, jnp.float32)

        rel_cp = pltpu.async_copy(rel_hbm, rtab, rsem)
        pltpu.sync_copy(e_hbm.at[wid], ev)
        pltpu.sync_copy(r_hbm.at[wid], rv)
        e_vecs = [ev[0, pl.ds(k * lanes, lanes)]
                  for k in range(bpw // lanes)]
        r_vecs = [rv[0, pl.ds(k * lanes, lanes)]
                  for k in range(bpw // lanes)]

        rows01 = iota % 2
        mcopies = [None] * bpw
        mlanes = [None] * bpw
        wids = [None] * (2 * bpw)

        def m_issue(i):
            e = e_vecs[i // lanes][i % lanes]
            et = pl.multiple_of((e // 128) * 128, 128)
            mlanes[i] = e - et
            mcopies[i] = pltpu.async_copy(
                map_hbm.at[:, pl.ds(et, 128)], mbufs[i % nbuf], msem)

        def m_extract(i):
            mcopies[i].wait()
            vals = plsc.load_gather(
                mbufs[i % nbuf],
                [rows01, jnp.full((lanes,), 0, jnp.int32) + mlanes[i]])
            wids[i] = vals[0]
            wids[bpw + i] = vals[1]

        ncols = 2 * bpw
        wcopies = [None] * ncols
        wlanes = [None] * ncols

        def w_issue(k, j):
            jt = pl.multiple_of((j // 128) * 128, 128)
            wlanes[k] = j - jt
            wcopies[k] = pltpu.async_copy(
                wt_hbm.at[:, pl.ds(jt, 128)], wbufs[k % nbuf], wsem)

        def w_extract(k):
            wcopies[k].wait()
            row0 = 0 if k % 2 == 0 else _HIDDEN
            col = jnp.full((lanes,), k // 2, jnp.int32)
            lane_idx = jnp.full((lanes,), 0, jnp.int32) + wlanes[k]
            for h in range(_HIDDEN // lanes):
                vals = plsc.load_gather(
                    wbufs[k % nbuf], [iota + h * lanes, lane_idx])
                plsc.store_scatter(
                    slab_v, [col, iota + (row0 + h * lanes)], vals)

        for i in range(min(nbuf, bpw)):
            m_issue(i)
        for i in range(bpw):
            m_extract(i)
            if i + nbuf < bpw:
                m_issue(i + nbuf)
            for k in (2 * i, 2 * i + 1):
                if k >= nbuf:
                    w_extract(k - nbuf)
                w_issue(k, wids[i] if k % 2 == 0 else wids[bpw + i])
        for k in range(ncols - nbuf, ncols):
            w_extract(k)

        rel_cp.wait()
        for i in range(bpw):
            rid = r_vecs[i // lanes][i % lanes]
            col = jnp.full((lanes,), i, jnp.int32)
            lane_idx = jnp.full((lanes,), 0, jnp.int32) + rid
            for h in range(_HIDDEN // lanes):
                vals = plsc.load_gather(rtab, [iota + h * lanes, lane_idx])
                plsc.store_scatter(
                    slab_v, [col, iota + (2 * _HIDDEN + h * lanes)], vals)

        ones_col = jnp.full((lanes,), 3 * _HIDDEN, jnp.int32)
        for k in range(bpw // lanes):
            plsc.store_scatter(slab_v, [iota + k * lanes, ones_col], ones16)

        pltpu.sync_copy(slab_v, mt_out.at[wid])

    return gather_k(e3, r3, emapT, wT, relT)


def _mm_body(wt_ref, b_ref, mt_ref, out_ref):
    lhs = jnp.concatenate([wt_ref[...], b_ref[...]], axis=0)
    out_ref[...] = lax.dot_general(
        lhs, mt_ref[...], (((0,), (0,)), ((), ())),
        preferred_element_type=jnp.float32)


def _tc_project(WT, b2d, mt_aug):
    grid = (pl.cdiv(_NUM_ENT, _BN),)
    outT = pl.pallas_call(
        _mm_body,
        grid=grid,
        in_specs=[
            pl.BlockSpec((3 * _HIDDEN, _BN), lambda i: (0, i)),
            pl.BlockSpec((1, _BN), lambda i: (0, i)),
            pl.BlockSpec((3 * _HIDDEN + 1, _BATCH), lambda i: (0, 0)),
        ],
        out_specs=pl.BlockSpec((_BN, _BATCH), lambda i: (i, 0)),
        out_shape=jax.ShapeDtypeStruct((_NUM_ENT, _BATCH), jnp.float32),
        compiler_params=pltpu.CompilerParams(
            dimension_semantics=("arbitrary",)),
    )(WT, b2d, mt_aug)
    return outT.T


def kernel(batch_data, ent_word_map, words_embd, rel_embed, W, b):
    nw = 32
    bpw = _BATCH // nw
    e3 = batch_data[:, 0].reshape(nw, 1, bpw)
    r3 = batch_data[:, 1].reshape(nw, 1, bpw)
    m3 = _sc_gather_all(e3, r3, ent_word_map.T, words_embd.T, rel_embed.T)
    mt_aug = m3.reshape(_BATCH, 3 * _HIDDEN + 1).T
    return _tc_project(W.T, b.reshape(1, -1), mt_aug)
